# baseline (speedup 1.0000x reference)
import functools

import jax
import jax.numpy as jnp
from jax import lax
from jax.experimental import pallas as pl
from jax.experimental.pallas import tpu as pltpu
from jax.experimental.pallas import tpu_sc as plsc

L = 16
PCTS = (0.25, 0.5, 0.75)


def _step(c, x, jf):
    csum, f0, f1, f2, a0, a1, a2, b0, b1, b2, i0, i1, i2 = c
    new = csum + x
    outs = []
    for p, f, a, b, i in ((PCTS[0], f0, a0, b0, i0),
                          (PCTS[1], f1, a1, b1, i1),
                          (PCTS[2], f2, a2, b2, i2)):
        newly = jnp.logical_and(jnp.logical_not(f), new >= p)
        outs.append((jnp.logical_or(f, newly),
                     jnp.where(newly, csum, a),
                     jnp.where(newly, new, b),
                     jnp.where(newly, jf, i)))
    (f0, a0, b0, i0), (f1, a1, b1, i1), (f2, a2, b2, i2) = outs
    return (new, f0, f1, f2, a0, a1, a2, b0, b1, b2, i0, i1, i2)


def _scan_group(hbm, tail, buf, deep, g, row0, n):
    lane = lax.iota(jnp.int32, L)
    lane16 = lane * L

    zf = jnp.zeros((L,), jnp.float32)
    nf = jnp.zeros((L,), jnp.bool_)

    c = (zf, nf, nf, nf, zf, zf, zf,
         jnp.ones((L,), jnp.float32), jnp.ones((L,), jnp.float32),
         jnp.ones((L,), jnp.float32), zf, zf, zf)

    cdf0 = None
    for j in range(L):
        flat = g * (L * L) + j + lane16
        x = plsc.load_gather(buf, [lax.shift_right_logical(flat, 7),
                                   lax.bitwise_and(flat, 127)])
        c = _step(c, x, float(j))
        if j == 0:
            cdf0 = c[0]

    def chunk_scan(k, c):
        nchunk = (n + 127) // 128
        if k == nchunk - 1:
            pltpu.sync_copy(tail.at[pl.ds(row0, L), :], deep)
        else:
            pltpu.sync_copy(hbm.at[pl.ds(row0, L), pl.ds(k * 128, 128)], deep)

        def body(jj, c):
            x = plsc.load_gather(deep, [lane, jnp.full((L,), jj, jnp.int32)])
            return _step(c, x, (k * 128 + jj).astype(jnp.float32))

        return lax.fori_loop(L if k == 0 else 0, 128, body, c)

    for k in range((n + 127) // 128):
        c = lax.cond(jnp.all(c[3]), lambda c: c,
                     functools.partial(chunk_scan, k), c)

    (_, f0, f1, f2, a0, a1, a2, b0, b1, b2, i0, i1, i2) = c

    qs = []
    for p, f, a, b, i in ((PCTS[0], f0, a0, b0, i0),
                          (PCTS[1], f1, a1, b1, i1),
                          (PCTS[2], f2, a2, b2, i2)):
        qs.append(jnp.where(f, i + 1.0 + (p - b) / (b - a), p / cdf0))
    return qs


def _scan_fast(buf, cbuf, g):
    lane = lax.iota(jnp.int32, L)
    lane16 = lane * L
    zi = jnp.zeros((L,), jnp.int32)

    def col(c, x, j):
        csum, c0, c1, c2 = c
        new = csum + x
        plsc.store_scatter(cbuf, [jnp.full((L,), j, jnp.int32), lane], new)
        c0 = c0 + (new < PCTS[0]).astype(jnp.int32)
        c1 = c1 + (new < PCTS[1]).astype(jnp.int32)
        c2 = c2 + (new < PCTS[2]).astype(jnp.int32)
        return (new, c0, c1, c2)

    c = (jnp.zeros((L,), jnp.float32), zi, zi, zi)
    for j in range(8):
        flat = g * (L * L) + j + lane16
        x = plsc.load_gather(buf, [lax.shift_right_logical(flat, 7),
                                   lax.bitwise_and(flat, 127)])
        c = col(c, x, j)

    def mid(c):
        def body(j, c):
            flat = g * (L * L) + j + lane16
            x = plsc.load_gather(buf, [lax.shift_right_logical(flat, 7),
                                       lax.bitwise_and(flat, 127)])
            return col(c, x, j)
        return lax.fori_loop(8, L, body, c)

    c = lax.cond(jnp.all(c[0] >= PCTS[2]), lambda c: c, mid, c)
    _, c0, c1, c2 = c
    bad = c2 >= L

    qs = []
    for p, cnt in ((PCTS[0], c0), (PCTS[1], c1), (PCTS[2], c2)):
        cc = jnp.minimum(cnt, L - 1)
        yb = plsc.load_gather(cbuf, [cc, lane])
        ya_prev = plsc.load_gather(cbuf, [jnp.maximum(cc - 1, 0), lane])
        ya = jnp.where(cnt == 0, jnp.zeros((L,), jnp.float32), ya_prev)
        qs.append(cnt.astype(jnp.float32) + 1.0 + (p - yb) / (yb - ya))
    return qs, bad


def _make_fast_call(B, N):
    rw = B // 32
    ng = rw // L
    srows = rw * L // 128
    mesh = plsc.VectorSubcoreMesh(core_axis_name="c", subcore_axis_name="s")

    @functools.partial(
        pl.kernel,
        mesh=mesh,
        out_type=jax.ShapeDtypeStruct((2, 32, L), jnp.float32),
        scratch_types=[
            pltpu.VMEM((srows, 128), jnp.float32),
            pltpu.VMEM((srows, 128), jnp.float32),
            pltpu.VMEM((L, L), jnp.float32),
            pltpu.VMEM((L, L), jnp.float32),
            pltpu.VMEM((L,), jnp.float32),
            pltpu.VMEM((L,), jnp.float32),
        ],
        compiler_params=pltpu.CompilerParams(needs_layout_passes=False),
    )
    def k(e16, t16, out, ebuf, tbuf, ecdf, tcdf, accv, badv):
        wid = lax.axis_index("c") * 16 + lax.axis_index("s")
        pltpu.sync_copy(e16.at[pl.ds(wid * srows, srows), :], ebuf)
        pltpu.sync_copy(t16.at[pl.ds(wid * srows, srows), :], tbuf)

        def group(g, carry):
            acc, badf = carry
            qt, badt = _scan_fast(tbuf, tcdf, g)
            qe, bade = _scan_fast(ebuf, ecdf, g)
            badrow = jnp.logical_or(badt, bade)
            s = jnp.zeros((L,), jnp.float32)
            for qti, qei in zip(qt, qe):
                s = s + jnp.abs(qti - qei)
            zf = jnp.zeros((L,), jnp.float32)
            acc = acc + jnp.where(badrow, zf, s)
            badf = badf + jnp.where(badrow, jnp.ones((L,), jnp.float32), zf)
            return acc, badf

        acc, badf = lax.fori_loop(
            0, ng, group,
            (jnp.zeros((L,), jnp.float32), jnp.zeros((L,), jnp.float32)))
        accv[...] = acc
        pltpu.sync_copy(accv, out.at[0, wid])
        badv[...] = badf
        pltpu.sync_copy(badv, out.at[1, wid])

    return k


def _make_sc_call(B, N):
    rw = B // 32
    ng = rw // L
    srows = rw * L // 128
    mesh = plsc.VectorSubcoreMesh(core_axis_name="c", subcore_axis_name="s")

    @functools.partial(
        pl.kernel,
        mesh=mesh,
        out_type=jax.ShapeDtypeStruct((32, L), jnp.float32),
        scratch_types=[
            pltpu.VMEM((srows, 128), jnp.float32),
            pltpu.VMEM((srows, 128), jnp.float32),
            pltpu.VMEM((L, 128), jnp.float32),
            pltpu.VMEM((L, 128), jnp.float32),
            pltpu.VMEM((L,), jnp.float32),
        ],
        compiler_params=pltpu.CompilerParams(needs_layout_passes=False),
    )
    def k(e16, t16, e_hbm, t_hbm, tail_e, tail_t, out,
          ebuf, tbuf, edeep, tdeep, accv):
        wid = lax.axis_index("c") * 16 + lax.axis_index("s")
        base = wid * rw
        pltpu.sync_copy(e16.at[pl.ds(wid * srows, srows), :], ebuf)
        pltpu.sync_copy(t16.at[pl.ds(wid * srows, srows), :], tbuf)

        def group(g, acc):
            row0 = base + g * L
            qt = _scan_group(t_hbm, tail_t, tbuf, tdeep, g, row0, N)
            qe = _scan_group(e_hbm, tail_e, ebuf, edeep, g, row0, N)
            for qti, qei in zip(qt, qe):
                acc = acc + jnp.abs(qti - qei)
            return acc

        acc = lax.fori_loop(0, ng, group, jnp.zeros((L,), jnp.float32))
        accv[...] = acc
        pltpu.sync_copy(accv, out.at[wid])

    return k


def _stage_body(e_ref, t_ref, eo_ref, to_ref):
    ro, rs = eo_ref.shape[0], e_ref.shape[0]
    i_out = lax.broadcasted_iota(jnp.int32, (ro, rs), 0)
    i_src = lax.broadcasted_iota(jnp.int32, (ro, rs), 1)
    for src, dst in ((e_ref, eo_ref), (t_ref, to_ref)):
        x = src[:, :L]
        pieces = []
        for k in range(8):
            sel = (i_src == i_out * 8 + k).astype(jnp.float32)
            pieces.append(jnp.dot(sel, x,
                                  preferred_element_type=jnp.float32))
        dst[...] = jnp.concatenate(pieces, axis=1)


def _stage_prefix(p_estimate, p_target):
    B = p_estimate.shape[0]
    R = 512
    sds = jax.ShapeDtypeStruct((B * L // 128, 128), jnp.float32)
    return pl.pallas_call(
        _stage_body,
        grid=(B // R,),
        in_specs=[pl.BlockSpec((R, 128), lambda i: (i, 0)),
                  pl.BlockSpec((R, 128), lambda i: (i, 0))],
        out_specs=[pl.BlockSpec((R * L // 128, 128), lambda i: (i, 0)),
                   pl.BlockSpec((R * L // 128, 128), lambda i: (i, 0))],
        out_shape=[sds, sds],
    )(p_estimate, p_target)


@jax.jit
def kernel(p_estimate, p_target):
    B, N = p_estimate.shape
    denom = jnp.float32(B * len(PCTS))
    e16, t16 = _stage_prefix(p_estimate, p_target)

    part = _make_fast_call(B, N)(e16, t16)
    nbad = jnp.sum(part[1])

    def deep(_):
        tail0 = 128 * ((N + 127) // 128 - 1)
        pad = 128 - (N - tail0)
        tail_e = jnp.pad(p_estimate[:, tail0:], ((0, 0), (0, pad)))
        tail_t = jnp.pad(p_target[:, tail0:], ((0, 0), (0, pad)))
        partial = _make_sc_call(B, N)(e16, t16, p_estimate, p_target,
                                      tail_e, tail_t)
        return jnp.sum(partial) / denom

    return lax.cond(nbad > 0, deep,
                    lambda _: jnp.sum(part[0]) / denom, None)


# speedup vs baseline: 3.3723x; 3.3723x over previous
import functools

import jax
import jax.numpy as jnp
from jax import lax
from jax.experimental import pallas as pl
from jax.experimental.pallas import tpu as pltpu
from jax.experimental.pallas import tpu_sc as plsc

L = 16
PCTS = (0.25, 0.5, 0.75)


def _step(c, x, jf):
    csum, f0, f1, f2, a0, a1, a2, b0, b1, b2, i0, i1, i2 = c
    new = csum + x
    outs = []
    for p, f, a, b, i in ((PCTS[0], f0, a0, b0, i0),
                          (PCTS[1], f1, a1, b1, i1),
                          (PCTS[2], f2, a2, b2, i2)):
        newly = jnp.logical_and(jnp.logical_not(f), new >= p)
        outs.append((jnp.logical_or(f, newly),
                     jnp.where(newly, csum, a),
                     jnp.where(newly, new, b),
                     jnp.where(newly, jf, i)))
    (f0, a0, b0, i0), (f1, a1, b1, i1), (f2, a2, b2, i2) = outs
    return (new, f0, f1, f2, a0, a1, a2, b0, b1, b2, i0, i1, i2)


def _scan_group(hbm, tail, buf, deep, g, row0, n):
    lane = lax.iota(jnp.int32, L)
    lane16 = lane * L

    zf = jnp.zeros((L,), jnp.float32)
    nf = jnp.zeros((L,), jnp.bool_)

    c = (zf, nf, nf, nf, zf, zf, zf,
         jnp.ones((L,), jnp.float32), jnp.ones((L,), jnp.float32),
         jnp.ones((L,), jnp.float32), zf, zf, zf)

    cdf0 = None
    for j in range(L):
        flat = g * (L * L) + j + lane16
        x = plsc.load_gather(buf, [lax.shift_right_logical(flat, 7),
                                   lax.bitwise_and(flat, 127)])
        c = _step(c, x, float(j))
        if j == 0:
            cdf0 = c[0]

    def chunk_scan(k, c):
        nchunk = (n + 127) // 128
        if k == nchunk - 1:
            pltpu.sync_copy(tail.at[pl.ds(row0, L), :], deep)
        else:
            pltpu.sync_copy(hbm.at[pl.ds(row0, L), pl.ds(k * 128, 128)], deep)

        def body(jj, c):
            x = plsc.load_gather(deep, [lane, jnp.full((L,), jj, jnp.int32)])
            return _step(c, x, (k * 128 + jj).astype(jnp.float32))

        return lax.fori_loop(L if k == 0 else 0, 128, body, c)

    for k in range((n + 127) // 128):
        c = lax.cond(jnp.all(c[3]), lambda c: c,
                     functools.partial(chunk_scan, k), c)

    (_, f0, f1, f2, a0, a1, a2, b0, b1, b2, i0, i1, i2) = c

    qs = []
    for p, f, a, b, i in ((PCTS[0], f0, a0, b0, i0),
                          (PCTS[1], f1, a1, b1, i1),
                          (PCTS[2], f2, a2, b2, i2)):
        qs.append(jnp.where(f, i + 1.0 + (p - b) / (b - a), p / cdf0))
    return qs


def _scan_fast(bufs, cbufs, g):
    lane = lax.iota(jnp.int32, L)
    lane16 = lane * L
    zi = jnp.zeros((L,), jnp.int32)

    def col_hot(cdf_ref, c, x, j):
        csum, c0, c1, c2 = c
        new = csum + x
        cdf_ref[j, :] = new
        c0 = c0 + (new < PCTS[0]).astype(jnp.int32)
        c1 = c1 + (new < PCTS[1]).astype(jnp.int32)
        c2 = c2 + (new < PCTS[2]).astype(jnp.int32)
        return (new, c0, c1, c2)

    def col_mid(cdf_ref, c, x, j):
        csum, c0, c1, c2 = c
        new = csum + x
        plsc.store_scatter(cdf_ref, [jnp.full((L,), j, jnp.int32), lane], new)
        c0 = c0 + (new < PCTS[0]).astype(jnp.int32)
        c1 = c1 + (new < PCTS[1]).astype(jnp.int32)
        c2 = c2 + (new < PCTS[2]).astype(jnp.int32)
        return (new, c0, c1, c2)

    c = ((jnp.zeros((L,), jnp.float32), zi, zi, zi),
         (jnp.zeros((L,), jnp.float32), zi, zi, zi))
    for j in range(8):
        flat = g * (L * L) + j + lane16
        r, cl = lax.shift_right_logical(flat, 7), lax.bitwise_and(flat, 127)
        xe = plsc.load_gather(bufs[0], [r, cl])
        xt = plsc.load_gather(bufs[1], [r, cl])
        c = (col_hot(cbufs[0], c[0], xe, j), col_hot(cbufs[1], c[1], xt, j))

    def mid(c):
        def body(j, c):
            flat = g * (L * L) + j + lane16
            r = lax.shift_right_logical(flat, 7)
            cl = lax.bitwise_and(flat, 127)
            xe = plsc.load_gather(bufs[0], [r, cl])
            xt = plsc.load_gather(bufs[1], [r, cl])
            return (col_mid(cbufs[0], c[0], xe, j),
                    col_mid(cbufs[1], c[1], xt, j))
        return lax.fori_loop(8, L, body, c)

    done = jnp.logical_and(jnp.all(c[0][0] >= PCTS[2]),
                           jnp.all(c[1][0] >= PCTS[2]))
    c = lax.cond(done, lambda c: c, mid, c)

    out = []
    for (_, c0, c1, c2), cbuf in zip(c, cbufs):
        bad = c2 >= L
        qs = []
        for p, cnt in ((PCTS[0], c0), (PCTS[1], c1), (PCTS[2], c2)):
            cc = jnp.minimum(cnt, L - 1)
            yb = plsc.load_gather(cbuf, [cc, lane])
            ya_prev = plsc.load_gather(cbuf, [jnp.maximum(cc - 1, 0), lane])
            ya = jnp.where(cnt == 0, jnp.zeros((L,), jnp.float32), ya_prev)
            qs.append(cnt.astype(jnp.float32) + 1.0 + (p - yb) / (yb - ya))
        out.append((qs, bad))
    return out


def _make_fast_call(B, N):
    rw = B // 32
    ng = rw // L
    srows = rw * L // 128
    mesh = plsc.VectorSubcoreMesh(core_axis_name="c", subcore_axis_name="s")

    @functools.partial(
        pl.kernel,
        mesh=mesh,
        out_type=jax.ShapeDtypeStruct((2, 32, L), jnp.float32),
        scratch_types=[
            pltpu.VMEM((srows, 128), jnp.float32),
            pltpu.VMEM((srows, 128), jnp.float32),
            pltpu.VMEM((L, L), jnp.float32),
            pltpu.VMEM((L, L), jnp.float32),
            pltpu.VMEM((L,), jnp.float32),
            pltpu.VMEM((L,), jnp.float32),
            pltpu.SemaphoreType.DMA,
        ],
        compiler_params=pltpu.CompilerParams(needs_layout_passes=False),
    )
    def k(e16, t16, out, ebuf, tbuf, ecdf, tcdf, accv, badv, sem):
        wid = lax.axis_index("c") * 16 + lax.axis_index("s")
        cp1 = pltpu.async_copy(e16.at[pl.ds(wid * srows, srows), :], ebuf,
                               sem)
        cp2 = pltpu.async_copy(t16.at[pl.ds(wid * srows, srows), :], tbuf,
                               sem)
        cp1.wait()
        cp2.wait()

        def group(g, carry):
            acc, badf = carry
            (qe, bade), (qt, badt) = _scan_fast((ebuf, tbuf), (ecdf, tcdf), g)
            badrow = jnp.logical_or(badt, bade)
            s = jnp.zeros((L,), jnp.float32)
            for qti, qei in zip(qt, qe):
                s = s + jnp.abs(qti - qei)
            zf = jnp.zeros((L,), jnp.float32)
            acc = acc + jnp.where(badrow, zf, s)
            badf = badf + jnp.where(badrow, jnp.ones((L,), jnp.float32), zf)
            return acc, badf

        acc, badf = lax.fori_loop(
            0, ng, group,
            (jnp.zeros((L,), jnp.float32), jnp.zeros((L,), jnp.float32)))
        accv[...] = acc
        pltpu.sync_copy(accv, out.at[0, wid])
        badv[...] = badf
        pltpu.sync_copy(badv, out.at[1, wid])

    return k


def _make_sc_call(B, N):
    rw = B // 32
    ng = rw // L
    srows = rw * L // 128
    mesh = plsc.VectorSubcoreMesh(core_axis_name="c", subcore_axis_name="s")

    @functools.partial(
        pl.kernel,
        mesh=mesh,
        out_type=jax.ShapeDtypeStruct((32, L), jnp.float32),
        scratch_types=[
            pltpu.VMEM((srows, 128), jnp.float32),
            pltpu.VMEM((srows, 128), jnp.float32),
            pltpu.VMEM((L, 128), jnp.float32),
            pltpu.VMEM((L, 128), jnp.float32),
            pltpu.VMEM((L,), jnp.float32),
        ],
        compiler_params=pltpu.CompilerParams(needs_layout_passes=False),
    )
    def k(e16, t16, e_hbm, t_hbm, tail_e, tail_t, out,
          ebuf, tbuf, edeep, tdeep, accv):
        wid = lax.axis_index("c") * 16 + lax.axis_index("s")
        base = wid * rw
        pltpu.sync_copy(e16.at[pl.ds(wid * srows, srows), :], ebuf)
        pltpu.sync_copy(t16.at[pl.ds(wid * srows, srows), :], tbuf)

        def group(g, acc):
            row0 = base + g * L
            qt = _scan_group(t_hbm, tail_t, tbuf, tdeep, g, row0, N)
            qe = _scan_group(e_hbm, tail_e, ebuf, edeep, g, row0, N)
            for qti, qei in zip(qt, qe):
                acc = acc + jnp.abs(qti - qei)
            return acc

        acc = lax.fori_loop(0, ng, group, jnp.zeros((L,), jnp.float32))
        accv[...] = acc
        pltpu.sync_copy(accv, out.at[wid])

    return k


@jax.jit
def kernel(p_estimate, p_target):
    B, N = p_estimate.shape
    denom = jnp.float32(B * len(PCTS))
    e16 = p_estimate[:, :L].reshape(B * L // 128, 128)
    t16 = p_target[:, :L].reshape(B * L // 128, 128)

    part = _make_fast_call(B, N)(e16, t16)
    nbad = jnp.sum(part[1])

    def deep(_):
        tail0 = 128 * ((N + 127) // 128 - 1)
        pad = 128 - (N - tail0)
        tail_e = jnp.pad(p_estimate[:, tail0:], ((0, 0), (0, pad)))
        tail_t = jnp.pad(p_target[:, tail0:], ((0, 0), (0, pad)))
        partial = _make_sc_call(B, N)(e16, t16, p_estimate, p_target,
                                      tail_e, tail_t)
        return jnp.sum(partial) / denom

    return lax.cond(nbad > 0, deep,
                    lambda _: jnp.sum(part[0]) / denom, None)
